# baseline (speedup 1.0000x reference)
import functools

import jax
import jax.numpy as jnp
from jax import lax
from jax.experimental import pallas as pl
from jax.experimental.pallas import tpu as pltpu
from jax.experimental.pallas import tpu_sc as plsc

N_NEIGH = 10
ACQ = 256
BK = 2048

_NC = 2
_NS = 16
_NW = _NC * _NS


def _extract_topn(vals, idxs, n):
    INF = jnp.float32(jnp.inf)
    BIGI = jnp.int32(2**30)
    res_v, res_i = [], []
    work = vals
    for _ in range(n):
        m = jnp.min(work, axis=1, keepdims=True)
        ismin = work == m
        sel = jnp.min(jnp.where(ismin, idxs, BIGI), axis=1, keepdims=True)
        res_v.append(m)
        res_i.append(sel)
        work = jnp.where(ismin & (idxs == sel), INF, work)
    return jnp.concatenate(res_v, axis=1), jnp.concatenate(res_i, axis=1)


def _topn_kernel(a_ref, b_ref, idx_ref, cv_ref, ci_ref, bv_ref, bi_ref,
                 wk_ref, *, n_blocks):
    Q = a_ref.shape[0]
    blk = pl.program_id(0)
    INF = jnp.float32(jnp.inf)
    BIGI = jnp.int32(2**30)

    @pl.when(blk == 0)
    def _init():
        cv_ref[...] = jnp.full(cv_ref.shape, INF, jnp.float32)
        ci_ref[...] = jnp.full(ci_ref.shape, BIGI, jnp.int32)

    a = a_ref[...]
    b = b_ref[...]
    hi = jax.lax.Precision.HIGHEST

    ab2 = jax.lax.dot_general(a, b, (((1,), (1,)), ((), ())),
                              preferred_element_type=jnp.float32)
    asum = jnp.sum(a * a, axis=1, keepdims=True)
    bn_row = jax.lax.dot_general(jnp.ones((1, a.shape[1]), jnp.float32),
                                 b * b, (((1,), (1,)), ((), ())),
                                 precision=hi,
                                 preferred_element_type=jnp.float32)
    z = bn_row - 2.0 * ab2
    t = cv_ref[:, N_NEIGH - 1:N_NEIGH]
    t2 = t - asum
    marg = 1e-4 * (1.0 + jnp.abs(t2) + asum)
    cand = z < (t2 + marg)
    itermax = jnp.max(jnp.sum(cand.astype(jnp.int32), axis=1))

    gidx = blk * BK + jax.lax.broadcasted_iota(jnp.int32, z.shape, 1)

    @pl.when(itermax == 1)
    def _single():
        s = (asum + bn_row) - 2.0 * ab2
        wk = jnp.where(cand, s, INF)
        m = jnp.min(wk, axis=1, keepdims=True)
        ismin = wk == m
        sel = jnp.min(jnp.where(ismin, gidx, BIGI), axis=1, keepdims=True)
        bv_ref[...] = jnp.concatenate(
            [m, jnp.full((Q, 15), INF, jnp.float32)], axis=1)
        bi_ref[...] = jnp.concatenate(
            [sel, jnp.full((Q, 15), BIGI, jnp.int32)], axis=1)

    @pl.when(itermax > 1)
    def _prep():
        s = (asum + bn_row) - 2.0 * ab2
        bv_ref[...] = jnp.full(bv_ref.shape, INF, jnp.float32)
        bi_ref[...] = jnp.full(bi_ref.shape, BIGI, jnp.int32)
        wk_ref[...] = jnp.where(cand, s, INF)

    for n in range(N_NEIGH):
        @pl.when(jnp.logical_and(n < itermax, itermax > 1))
        def _it(n=n):
            work = wk_ref[...]
            m = jnp.min(work, axis=1, keepdims=True)
            ismin = work == m
            sel = jnp.min(jnp.where(ismin, gidx, BIGI), axis=1, keepdims=True)
            bv_ref[:, n:n + 1] = m
            bi_ref[:, n:n + 1] = sel
            wk_ref[...] = jnp.where(ismin & (gidx == sel), INF, work)

    @pl.when(itermax > 0)
    def _merge():
        cat_v = jnp.concatenate([cv_ref[...], bv_ref[...]], axis=1)
        cat_i = jnp.concatenate([ci_ref[...], bi_ref[...]], axis=1)
        nv, ni = _extract_topn(cat_v, cat_i, N_NEIGH)
        pad = cv_ref.shape[1] - N_NEIGH
        cv_ref[...] = jnp.concatenate(
            [nv, jnp.full((Q, pad), INF, jnp.float32)], axis=1)
        ci_ref[...] = jnp.concatenate(
            [ni, jnp.full((Q, pad), BIGI, jnp.int32)], axis=1)

    @pl.when(blk == n_blocks - 1)
    def _fin():
        idx_ref[...] = ci_ref[...]


def _sc_score_body(plp_hbm, idx_hbm, pul_hbm, out_hbm,
                   idx_v, rows_v, pu_v, out_v, sem, *, qpw):
    wid = lax.axis_index("s") * _NC + lax.axis_index("c")
    base = wid * qpw
    ipw = qpw * N_NEIGH
    pltpu.sync_copy(idx_hbm.at[pl.ds(base * N_NEIGH, ipw)], idx_v)
    pltpu.sync_copy(pul_hbm.at[pl.ds(base, qpw)], pu_v)
    chunk = 80
    for j in range(ipw // chunk):
        sl = pl.ds(j * chunk, chunk)
        pltpu.async_copy(plp_hbm.at[idx_v.at[sl]], rows_v.at[sl], sem).wait()

    for i in range(qpw):
        pu_log_row = pu_v[i, :]
        acc = jnp.zeros((16,), jnp.float32)
        for n in range(N_NEIGH):
            r = rows_v[i * N_NEIGH + n, :]
            pn = jnp.exp(r)
            acc = acc + (pn * (r - pu_log_row) - pn)
        out_v[i, :] = acc
    pltpu.sync_copy(out_v, out_hbm.at[pl.ds(base, qpw)])


def _score_kernel(acc_ref, pul_ref, score_ref):
    acc = acc_ref[...]
    pul = pul_ref[...]
    cq = jnp.sum(jnp.exp(pul), axis=1, keepdims=True)
    hsum = jnp.sum(acc, axis=1, keepdims=True)
    score_ref[...] = -(hsum * (1.0 / N_NEIGH) + cq)


def _topk_kernel(sc_ref, sr_ref, vals_ref, idx_ref):
    N = sc_ref.shape[0]
    scol = sc_ref[...]
    srow = sr_ref[...]
    ii = jax.lax.broadcasted_iota(jnp.int32, (N, N), 0)
    jj = jax.lax.broadcasted_iota(jnp.int32, (N, N), 1)
    beats = (srow > scol) | ((srow == scol) & (jj < ii))
    rank = jnp.sum(beats.astype(jnp.int32), axis=1, keepdims=True)
    rr = jax.lax.broadcasted_iota(jnp.int32, (N, ACQ), 1)
    oh = rank == rr
    qi = jax.lax.broadcasted_iota(jnp.int32, (N, ACQ), 0)
    vals_ref[...] = jnp.sum(jnp.where(oh, scol, 0.0), axis=0, keepdims=True)
    idx_ref[...] = jnp.sum(jnp.where(oh, qi, 0), axis=0, keepdims=True)


def kernel(z_unlab, z_lab, p_lab_log, p_unlab_log, acq_size):
    a = z_unlab[..., 0]
    b = z_lab[..., 1]
    Q, D = a.shape
    K = b.shape[0]
    C = p_lab_log.shape[1]
    nb = -(-K // BK)
    Kp = nb * BK
    bp = (jnp.pad(b, ((0, Kp - K), (0, 0)), constant_values=1e15)
          if Kp != K else b)

    idx16 = pl.pallas_call(
        functools.partial(_topn_kernel, n_blocks=nb),
        grid=(nb,),
        in_specs=[
            pl.BlockSpec((Q, D), lambda i: (0, 0)),
            pl.BlockSpec((BK, D), lambda i: (i, 0)),
        ],
        out_specs=pl.BlockSpec((Q, 16), lambda i: (0, 0)),
        out_shape=jax.ShapeDtypeStruct((Q, 16), jnp.int32),
        scratch_shapes=[
            pltpu.VMEM((Q, 16), jnp.float32),
            pltpu.VMEM((Q, 16), jnp.int32),
            pltpu.VMEM((Q, 16), jnp.float32),
            pltpu.VMEM((Q, 16), jnp.int32),
            pltpu.VMEM((Q, BK), jnp.float32),
        ],
    )(a, bp)

    idx10 = idx16[:, :N_NEIGH].reshape(-1)

    qpw = Q // _NW
    ipw = qpw * N_NEIGH
    sc_score = functools.partial(
        pl.kernel,
        mesh=plsc.VectorSubcoreMesh(core_axis_name="c", subcore_axis_name="s"),
        out_type=jax.ShapeDtypeStruct((Q, C), jnp.float32),
        scratch_types=[
            pltpu.VMEM((ipw,), jnp.int32),
            pltpu.VMEM((ipw, C), jnp.float32),
            pltpu.VMEM((qpw, C), jnp.float32),
            pltpu.VMEM((qpw, C), jnp.float32),
            pltpu.SemaphoreType.DMA,
        ],
        compiler_params=pltpu.CompilerParams(use_tc_tiling_on_sc=False),
    )(functools.partial(_sc_score_body, qpw=qpw))
    acc16 = sc_score(p_lab_log, idx10, p_unlab_log)

    score = pl.pallas_call(
        _score_kernel,
        out_shape=jax.ShapeDtypeStruct((Q, 1), jnp.float32),
    )(acc16, p_unlab_log)

    vals, idxs = pl.pallas_call(
        _topk_kernel,
        out_shape=(jax.ShapeDtypeStruct((1, ACQ), jnp.float32),
                   jax.ShapeDtypeStruct((1, ACQ), jnp.int32)),
    )(score.reshape(Q, 1), score.reshape(1, Q))
    return vals.reshape(ACQ), idxs.reshape(ACQ)


# speedup vs baseline: 1.1291x; 1.1291x over previous
import functools

import jax
import jax.numpy as jnp
from jax import lax
from jax.experimental import pallas as pl
from jax.experimental.pallas import tpu as pltpu
from jax.experimental.pallas import tpu_sc as plsc

N_NEIGH = 10
ACQ = 256
BK = 4096

_NC = 2
_NS = 16
_NW = _NC * _NS


def _extract_topn(vals, idxs, n):
    INF = jnp.float32(jnp.inf)
    BIGI = jnp.int32(2**30)
    res_v, res_i = [], []
    work = vals
    for _ in range(n):
        m = jnp.min(work, axis=1, keepdims=True)
        ismin = work == m
        sel = jnp.min(jnp.where(ismin, idxs, BIGI), axis=1, keepdims=True)
        res_v.append(m)
        res_i.append(sel)
        work = jnp.where(ismin & (idxs == sel), INF, work)
    return jnp.concatenate(res_v, axis=1), jnp.concatenate(res_i, axis=1)


def _topn_kernel(a_ref, b_ref, idx_ref, cv_ref, ci_ref, bv_ref, bi_ref,
                 wk_ref, *, n_blocks):
    Q = a_ref.shape[0]
    blk = pl.program_id(0)
    INF = jnp.float32(jnp.inf)
    BIGI = jnp.int32(2**30)

    @pl.when(blk == 0)
    def _init():
        cv_ref[...] = jnp.full(cv_ref.shape, INF, jnp.float32)
        ci_ref[...] = jnp.full(ci_ref.shape, BIGI, jnp.int32)

    a = a_ref[...]
    b = b_ref[...]
    hi = jax.lax.Precision.HIGHEST

    ab2 = jax.lax.dot_general(a, b, (((1,), (1,)), ((), ())),
                              preferred_element_type=jnp.float32)
    asum = jnp.sum(a * a, axis=1, keepdims=True)
    bn_row = jax.lax.dot_general(jnp.ones((1, a.shape[1]), jnp.float32),
                                 b * b, (((1,), (1,)), ((), ())),
                                 precision=hi,
                                 preferred_element_type=jnp.float32)
    s = (asum + bn_row) - 2.0 * ab2

    t = cv_ref[:, N_NEIGH - 1:N_NEIGH]
    cand = s < t
    itermax = jnp.max(jnp.sum(cand.astype(jnp.int32), axis=1))

    @pl.when(itermax > 0)
    def _prep():
        bv_ref[...] = jnp.full(bv_ref.shape, INF, jnp.float32)
        bi_ref[...] = jnp.full(bi_ref.shape, BIGI, jnp.int32)
        wk_ref[...] = jnp.where(cand, s, INF)

    gidx = blk * BK + jax.lax.broadcasted_iota(jnp.int32, s.shape, 1)
    for n in range(N_NEIGH):
        @pl.when(n < itermax)
        def _it(n=n):
            work = wk_ref[...]
            m = jnp.min(work, axis=1, keepdims=True)
            ismin = work == m
            sel = jnp.min(jnp.where(ismin, gidx, BIGI), axis=1, keepdims=True)
            bv_ref[:, n:n + 1] = m
            bi_ref[:, n:n + 1] = sel
            wk_ref[...] = jnp.where(ismin & (gidx == sel), INF, work)

    @pl.when(itermax > 0)
    def _merge():
        cat_v = jnp.concatenate([cv_ref[...], bv_ref[...]], axis=1)
        cat_i = jnp.concatenate([ci_ref[...], bi_ref[...]], axis=1)
        nv, ni = _extract_topn(cat_v, cat_i, N_NEIGH)
        pad = cv_ref.shape[1] - N_NEIGH
        cv_ref[...] = jnp.concatenate(
            [nv, jnp.full((Q, pad), INF, jnp.float32)], axis=1)
        ci_ref[...] = jnp.concatenate(
            [ni, jnp.full((Q, pad), BIGI, jnp.int32)], axis=1)

    @pl.when(blk == n_blocks - 1)
    def _fin():
        idx_ref[...] = ci_ref[...]


def _sc_score_body(plp_hbm, idx_hbm, pul_hbm, out_hbm,
                   idx_v, rows_v, pu_v, out_v, sem, *, qpw):
    wid = lax.axis_index("s") * _NC + lax.axis_index("c")
    base = wid * qpw
    ipw = qpw * N_NEIGH
    pltpu.sync_copy(idx_hbm.at[pl.ds(base * N_NEIGH, ipw)], idx_v)
    pltpu.sync_copy(pul_hbm.at[pl.ds(base, qpw)], pu_v)
    chunk = 80
    for j in range(ipw // chunk):
        sl = pl.ds(j * chunk, chunk)
        pltpu.async_copy(plp_hbm.at[idx_v.at[sl]], rows_v.at[sl], sem).wait()

    for i in range(qpw):
        pu_log_row = pu_v[i, :]
        acc = jnp.zeros((16,), jnp.float32)
        for n in range(N_NEIGH):
            r = rows_v[i * N_NEIGH + n, :]
            pn = jnp.exp(r)
            acc = acc + (pn * (r - pu_log_row) - pn)
        out_v[i, :] = acc
    pltpu.sync_copy(out_v, out_hbm.at[pl.ds(base, qpw)])


def _score_kernel(acc_ref, pul_ref, score_ref):
    acc = acc_ref[...]
    pul = pul_ref[...]
    cq = jnp.sum(jnp.exp(pul), axis=1, keepdims=True)
    hsum = jnp.sum(acc, axis=1, keepdims=True)
    score_ref[...] = -(hsum * (1.0 / N_NEIGH) + cq)


def _topk_kernel(sc_ref, sr_ref, vals_ref, idx_ref):
    N = sc_ref.shape[0]
    scol = sc_ref[...]
    srow = sr_ref[...]
    ii = jax.lax.broadcasted_iota(jnp.int32, (N, N), 0)
    jj = jax.lax.broadcasted_iota(jnp.int32, (N, N), 1)
    beats = (srow > scol) | ((srow == scol) & (jj < ii))
    rank = jnp.sum(beats.astype(jnp.int32), axis=1, keepdims=True)
    rr = jax.lax.broadcasted_iota(jnp.int32, (N, ACQ), 1)
    oh = rank == rr
    qi = jax.lax.broadcasted_iota(jnp.int32, (N, ACQ), 0)
    vals_ref[...] = jnp.sum(jnp.where(oh, scol, 0.0), axis=0, keepdims=True)
    idx_ref[...] = jnp.sum(jnp.where(oh, qi, 0), axis=0, keepdims=True)


def kernel(z_unlab, z_lab, p_lab_log, p_unlab_log, acq_size):
    a = z_unlab[..., 0]
    b = z_lab[..., 1]
    Q, D = a.shape
    K = b.shape[0]
    C = p_lab_log.shape[1]
    nb = -(-K // BK)
    Kp = nb * BK
    bp = (jnp.pad(b, ((0, Kp - K), (0, 0)), constant_values=1e15)
          if Kp != K else b)

    idx16 = pl.pallas_call(
        functools.partial(_topn_kernel, n_blocks=nb),
        grid=(nb,),
        in_specs=[
            pl.BlockSpec((Q, D), lambda i: (0, 0)),
            pl.BlockSpec((BK, D), lambda i: (i, 0)),
        ],
        out_specs=pl.BlockSpec((Q, 16), lambda i: (0, 0)),
        out_shape=jax.ShapeDtypeStruct((Q, 16), jnp.int32),
        scratch_shapes=[
            pltpu.VMEM((Q, 16), jnp.float32),
            pltpu.VMEM((Q, 16), jnp.int32),
            pltpu.VMEM((Q, 16), jnp.float32),
            pltpu.VMEM((Q, 16), jnp.int32),
            pltpu.VMEM((Q, BK), jnp.float32),
        ],
    )(a, bp)

    idx10 = idx16[:, :N_NEIGH].reshape(-1)

    qpw = Q // _NW
    ipw = qpw * N_NEIGH
    sc_score = functools.partial(
        pl.kernel,
        mesh=plsc.VectorSubcoreMesh(core_axis_name="c", subcore_axis_name="s"),
        out_type=jax.ShapeDtypeStruct((Q, C), jnp.float32),
        scratch_types=[
            pltpu.VMEM((ipw,), jnp.int32),
            pltpu.VMEM((ipw, C), jnp.float32),
            pltpu.VMEM((qpw, C), jnp.float32),
            pltpu.VMEM((qpw, C), jnp.float32),
            pltpu.SemaphoreType.DMA,
        ],
        compiler_params=pltpu.CompilerParams(use_tc_tiling_on_sc=False),
    )(functools.partial(_sc_score_body, qpw=qpw))
    acc16 = sc_score(p_lab_log, idx10, p_unlab_log)

    score = pl.pallas_call(
        _score_kernel,
        out_shape=jax.ShapeDtypeStruct((Q, 1), jnp.float32),
    )(acc16, p_unlab_log)

    vals, idxs = pl.pallas_call(
        _topk_kernel,
        out_shape=(jax.ShapeDtypeStruct((1, ACQ), jnp.float32),
                   jax.ShapeDtypeStruct((1, ACQ), jnp.int32)),
    )(score.reshape(Q, 1), score.reshape(1, Q))
    return vals.reshape(ACQ), idxs.reshape(ACQ)


# speedup vs baseline: 1.6366x; 1.4495x over previous
import functools

import jax
import jax.numpy as jnp
from jax import lax
from jax.experimental import pallas as pl
from jax.experimental.pallas import tpu as pltpu
from jax.experimental.pallas import tpu_sc as plsc

N_NEIGH = 10
ACQ = 256
BK = 2048

_NC = 2
_NS = 16
_NW = _NC * _NS


def _extract_topn(vals, idxs, n):
    INF = jnp.float32(jnp.inf)
    BIGI = jnp.int32(2**30)
    res_v, res_i = [], []
    work = vals
    for _ in range(n):
        m = jnp.min(work, axis=1, keepdims=True)
        ismin = work == m
        sel = jnp.min(jnp.where(ismin, idxs, BIGI), axis=1, keepdims=True)
        res_v.append(m)
        res_i.append(sel)
        work = jnp.where(ismin & (idxs == sel), INF, work)
    return jnp.concatenate(res_v, axis=1), jnp.concatenate(res_i, axis=1)


def _topn_kernel(a_ref, b_ref, idx_ref, cv_ref, ci_ref, bv_ref, bi_ref,
                 wk_ref, *, n_blocks):
    Q = a_ref.shape[0]
    blk = pl.program_id(0)
    INF = jnp.float32(jnp.inf)
    BIGI = jnp.int32(2**30)

    @pl.when(blk == 0)
    def _init():
        cv_ref[...] = jnp.full(cv_ref.shape, INF, jnp.float32)
        ci_ref[...] = jnp.full(ci_ref.shape, BIGI, jnp.int32)

    a = a_ref[...]
    b = b_ref[...]
    hi = jax.lax.Precision.HIGHEST

    ab2 = jax.lax.dot_general(a, b, (((1,), (1,)), ((), ())),
                              preferred_element_type=jnp.float32)
    asum = jnp.sum(a * a, axis=1, keepdims=True)
    bn_row = jax.lax.dot_general(jnp.ones((1, a.shape[1]), jnp.float32),
                                 b * b, (((1,), (1,)), ((), ())),
                                 precision=hi,
                                 preferred_element_type=jnp.float32)
    s = (asum + bn_row) - 2.0 * ab2

    t = cv_ref[:, N_NEIGH - 1:N_NEIGH]
    cand = s < t
    itermax = jnp.max(jnp.sum(cand.astype(jnp.int32), axis=1))

    @pl.when(itermax > 0)
    def _prep():
        bv_ref[...] = jnp.full(bv_ref.shape, INF, jnp.float32)
        bi_ref[...] = jnp.full(bi_ref.shape, BIGI, jnp.int32)
        wk_ref[...] = jnp.where(cand, s, INF)

    gidx = blk * BK + jax.lax.broadcasted_iota(jnp.int32, s.shape, 1)
    for n in range(N_NEIGH):
        @pl.when(n < itermax)
        def _it(n=n):
            work = wk_ref[...]
            m = jnp.min(work, axis=1, keepdims=True)
            ismin = work == m
            sel = jnp.min(jnp.where(ismin, gidx, BIGI), axis=1, keepdims=True)
            bv_ref[:, n:n + 1] = m
            bi_ref[:, n:n + 1] = sel
            wk_ref[...] = jnp.where(ismin & (gidx == sel), INF, work)

    @pl.when(itermax > 0)
    def _merge():
        cat_v = jnp.concatenate([cv_ref[...], bv_ref[...]], axis=1)
        cat_i = jnp.concatenate([ci_ref[...], bi_ref[...]], axis=1)
        nv, ni = _extract_topn(cat_v, cat_i, N_NEIGH)
        pad = cv_ref.shape[1] - N_NEIGH
        cv_ref[...] = jnp.concatenate(
            [nv, jnp.full((Q, pad), INF, jnp.float32)], axis=1)
        ci_ref[...] = jnp.concatenate(
            [ni, jnp.full((Q, pad), BIGI, jnp.int32)], axis=1)

    @pl.when(blk == n_blocks - 1)
    def _fin():
        idx_ref[...] = ci_ref[...]


def _sc_score_body(plp_hbm, idx_hbm, pul_hbm, out_hbm,
                   idx_v, rows_v, pu_v, out_v, sem, *, qpw):
    wid = lax.axis_index("s") * _NC + lax.axis_index("c")
    base = wid * qpw
    ipw = qpw * N_NEIGH
    pltpu.sync_copy(idx_hbm.at[pl.ds(base * N_NEIGH, ipw)], idx_v)
    pltpu.sync_copy(pul_hbm.at[pl.ds(base, qpw)], pu_v)
    chunk = 80
    for j in range(ipw // chunk):
        sl = pl.ds(j * chunk, chunk)
        pltpu.async_copy(plp_hbm.at[idx_v.at[sl]], rows_v.at[sl], sem).wait()

    for i in range(qpw):
        pu_log_row = pu_v[i, :]
        acc = jnp.zeros((16,), jnp.float32)
        for n in range(N_NEIGH):
            r = rows_v[i * N_NEIGH + n, :]
            pn = jnp.exp(r)
            acc = acc + (pn * (r - pu_log_row) - pn)
        out_v[i, :] = acc
    pltpu.sync_copy(out_v, out_hbm.at[pl.ds(base, qpw)])


def _score_kernel(acc_ref, pul_ref, score_ref):
    acc = acc_ref[...]
    pul = pul_ref[...]
    cq = jnp.sum(jnp.exp(pul), axis=1, keepdims=True)
    hsum = jnp.sum(acc, axis=1, keepdims=True)
    score_ref[...] = -(hsum * (1.0 / N_NEIGH) + cq)


def _topk_kernel(sc_ref, sr_ref, vals_ref, idx_ref):
    N = sc_ref.shape[0]
    scol = sc_ref[...]
    srow = sr_ref[...]
    ii = jax.lax.broadcasted_iota(jnp.int32, (N, N), 0)
    jj = jax.lax.broadcasted_iota(jnp.int32, (N, N), 1)
    beats = (srow > scol) | ((srow == scol) & (jj < ii))
    rank = jnp.sum(beats.astype(jnp.int32), axis=1, keepdims=True)
    rr = jax.lax.broadcasted_iota(jnp.int32, (N, ACQ), 1)
    oh = rank == rr
    qi = jax.lax.broadcasted_iota(jnp.int32, (N, ACQ), 0)
    vals_ref[...] = jnp.sum(jnp.where(oh, scol, 0.0), axis=0, keepdims=True)
    idx_ref[...] = jnp.sum(jnp.where(oh, qi, 0), axis=0, keepdims=True)


def kernel(z_unlab, z_lab, p_lab_log, p_unlab_log, acq_size):
    a = z_unlab[..., 0]
    b = z_lab[..., 1]
    Q, D = a.shape
    K = b.shape[0]
    C = p_lab_log.shape[1]
    nb = -(-K // BK)
    Kp = nb * BK
    bp = (jnp.pad(b, ((0, Kp - K), (0, 0)), constant_values=1e15)
          if Kp != K else b)

    idx16 = pl.pallas_call(
        functools.partial(_topn_kernel, n_blocks=nb),
        grid=(nb,),
        in_specs=[
            pl.BlockSpec((Q, D), lambda i: (0, 0)),
            pl.BlockSpec((BK, D), lambda i: (i, 0)),
        ],
        out_specs=pl.BlockSpec((Q, 16), lambda i: (0, 0)),
        out_shape=jax.ShapeDtypeStruct((Q, 16), jnp.int32),
        scratch_shapes=[
            pltpu.VMEM((Q, 16), jnp.float32),
            pltpu.VMEM((Q, 16), jnp.int32),
            pltpu.VMEM((Q, 16), jnp.float32),
            pltpu.VMEM((Q, 16), jnp.int32),
            pltpu.VMEM((Q, BK), jnp.float32),
        ],
    )(a, bp)

    idx10 = idx16[:, :N_NEIGH].reshape(-1)

    qpw = Q // _NW
    ipw = qpw * N_NEIGH
    sc_score = functools.partial(
        pl.kernel,
        mesh=plsc.VectorSubcoreMesh(core_axis_name="c", subcore_axis_name="s"),
        out_type=jax.ShapeDtypeStruct((Q, C), jnp.float32),
        scratch_types=[
            pltpu.VMEM((ipw,), jnp.int32),
            pltpu.VMEM((ipw, C), jnp.float32),
            pltpu.VMEM((qpw, C), jnp.float32),
            pltpu.VMEM((qpw, C), jnp.float32),
            pltpu.SemaphoreType.DMA,
        ],
        compiler_params=pltpu.CompilerParams(use_tc_tiling_on_sc=False),
    )(functools.partial(_sc_score_body, qpw=qpw))
    acc16 = sc_score(p_lab_log, idx10, p_unlab_log)

    score = pl.pallas_call(
        _score_kernel,
        out_shape=jax.ShapeDtypeStruct((Q, 1), jnp.float32),
    )(acc16, p_unlab_log)

    vals, idxs = pl.pallas_call(
        _topk_kernel,
        out_shape=(jax.ShapeDtypeStruct((1, ACQ), jnp.float32),
                   jax.ShapeDtypeStruct((1, ACQ), jnp.int32)),
    )(score.reshape(Q, 1), score.reshape(1, Q))
    return vals.reshape(ACQ), idxs.reshape(ACQ)
